# baseline (speedup 1.0000x reference)
import functools

import jax
import jax.numpy as jnp
from jax import lax
from jax.experimental import pallas as pl
from jax.experimental.pallas import tpu as pltpu
from jax.experimental.pallas import tpu_sc as plsc

_NEG = -1e30
_VBLK = 512


def _gather_rows(table, idx):
    info = plsc.get_sparse_core_info()
    nc, ns = info.num_cores, info.num_subcores
    nw = nc * ns
    b, d = idx.shape[0], table.shape[1]
    bw = b // nw
    mesh = plsc.VectorSubcoreMesh(core_axis_name="c", subcore_axis_name="s")

    @functools.partial(
        pl.kernel,
        mesh=mesh,
        out_type=jax.ShapeDtypeStruct((b, d), jnp.float32),
        scratch_types=[
            pltpu.VMEM((bw,), jnp.int32),
            pltpu.VMEM((bw, d), jnp.float32),
            pltpu.SemaphoreType.DMA,
        ],
        compiler_params=pltpu.CompilerParams(use_tc_tiling_on_sc=False),
    )
    def k(table_hbm, idx_hbm, out_hbm, idx_v, rows_v, sem):
        wid = lax.axis_index("s") * nc + lax.axis_index("c")
        base = wid * bw
        pltpu.sync_copy(idx_hbm.at[pl.ds(base, bw)], idx_v)
        pltpu.async_copy(table_hbm.at[idx_v], rows_v, sem).wait()
        pltpu.sync_copy(rows_v, out_hbm.at[pl.ds(base, bw)])

    return k(table, idx)


def _stats_body(vocab, vblk,
                x_ref, w1_ref, b1_ref, w2_ref, b2_ref,
                h_ref, lse_ref, m_ref, s_ref):
    v = pl.program_id(0)

    @pl.when(v == 0)
    def _():
        h_ref[...] = jnp.maximum(
            jnp.dot(x_ref[...], w1_ref[...], preferred_element_type=jnp.float32)
            + b1_ref[...], 0.0)

    logits = (jnp.dot(h_ref[...], w2_ref[...], preferred_element_type=jnp.float32)
              + b2_ref[...])
    cols = v * vblk + lax.broadcasted_iota(jnp.int32, logits.shape, 1)
    logits = jnp.where(cols < vocab, logits, _NEG)
    tm = jnp.max(logits, axis=1, keepdims=True)

    @pl.when(v == 0)
    def _():
        m_ref[...] = tm
        s_ref[...] = jnp.sum(jnp.exp(logits - tm), axis=1, keepdims=True)

    @pl.when(v > 0)
    def _():
        m_old = m_ref[...]
        m_new = jnp.maximum(m_old, tm)
        s_ref[...] = (s_ref[...] * jnp.exp(m_old - m_new)
                      + jnp.sum(jnp.exp(logits - m_new), axis=1, keepdims=True))
        m_ref[...] = m_new

    @pl.when(v == pl.num_programs(0) - 1)
    def _():
        lse_ref[...] = m_ref[...] + jnp.log(s_ref[...])


def _emit_body(h_ref, w2_ref, b2_ref, lse_ref, o_ref):
    o_ref[...] = (jnp.dot(h_ref[...], w2_ref[...], preferred_element_type=jnp.float32)
                  + b2_ref[...] - lse_ref[...])


def _mlp_logsoftmax(x, w1p, b1, w2, b2):
    bsz, k = x.shape
    hid = w1p.shape[1]
    vocab = w2.shape[1]
    nv = pl.cdiv(vocab, _VBLK)
    b1r = b1.reshape(1, hid)
    b2r = b2.reshape(1, vocab)
    h, lse = pl.pallas_call(
        functools.partial(_stats_body, vocab, _VBLK),
        grid=(nv,),
        in_specs=[
            pl.BlockSpec((bsz, k), lambda v: (0, 0)),
            pl.BlockSpec((k, hid), lambda v: (0, 0)),
            pl.BlockSpec((1, hid), lambda v: (0, 0)),
            pl.BlockSpec((hid, _VBLK), lambda v: (0, v)),
            pl.BlockSpec((1, _VBLK), lambda v: (0, v)),
        ],
        out_specs=[
            pl.BlockSpec((bsz, hid), lambda v: (0, 0)),
            pl.BlockSpec((bsz, 1), lambda v: (0, 0)),
        ],
        out_shape=[
            jax.ShapeDtypeStruct((bsz, hid), jnp.float32),
            jax.ShapeDtypeStruct((bsz, 1), jnp.float32),
        ],
        scratch_shapes=[
            pltpu.VMEM((bsz, 1), jnp.float32),
            pltpu.VMEM((bsz, 1), jnp.float32),
        ],
    )(x, w1p, b1r, w2, b2r)
    return pl.pallas_call(
        _emit_body,
        grid=(nv,),
        in_specs=[
            pl.BlockSpec((bsz, hid), lambda v: (0, 0)),
            pl.BlockSpec((hid, _VBLK), lambda v: (0, v)),
            pl.BlockSpec((1, _VBLK), lambda v: (0, v)),
            pl.BlockSpec((bsz, 1), lambda v: (0, 0)),
        ],
        out_specs=pl.BlockSpec((bsz, _VBLK), lambda v: (0, v)),
        out_shape=jax.ShapeDtypeStruct((bsz, vocab), jnp.float32),
    )(h, w2, b2r, lse)


def kernel(inputs, batch_size, emb, W1, b1, W2, b2):
    bsz, nctx = inputs.shape
    m = emb.shape[1]
    hid = W1.shape[1]
    mp = -(-m // 16) * 16
    table = jnp.pad(emb, ((0, 0), (0, mp - m)))
    idx = inputs.reshape(-1).astype(jnp.int32)
    rows = _gather_rows(table, idx)
    x = rows.reshape(bsz, nctx * mp)
    w1p = jnp.pad(W1.reshape(nctx, m, hid),
                  ((0, 0), (0, mp - m), (0, 0))).reshape(nctx * mp, hid)
    return _mlp_logsoftmax(x, w1p, b1, W2, b2)
